# Initial kernel scaffold; baseline (speedup 1.0000x reference)
import jax
import jax.numpy as jnp
from jax.experimental import pallas as pl


def kernel(x, Wr, W1, b1, W2, b2):
    raise NotImplementedError("write your pallas kernel here")


# speedup vs baseline: 1.7146x; 1.7146x over previous
import functools
import jax
import jax.numpy as jnp
from jax.experimental import pallas as pl
from jax.experimental.pallas import tpu as pltpu

_D, _F, _E = 1024, 2048, 8
_N = 2048
_TB = 512
_FC = 512
_NT = _N // _TB
_NF = _F // _FC
_AUX_W = 0.01


def _router_body(x_ref, wr_ref, we_ref, aux_ref):
    xl = x_ref[...]
    iota = jax.lax.broadcasted_iota(jnp.int32, (_N, 128), 1).astype(jnp.float32)
    logits = jnp.dot(xl, wr_ref[...], preferred_element_type=jnp.float32)
    logits = jnp.where(iota < _E, logits, -jnp.inf)
    m0 = jnp.max(logits, axis=1, keepdims=True)
    is0 = (logits == m0).astype(jnp.float32)
    i0 = 7.0 - jnp.max(is0 * (7.0 - iota) - (1.0 - is0) * 1e9, axis=1, keepdims=True)
    oh0 = (iota == i0).astype(jnp.float32)
    masked = jnp.where(oh0 > 0, -jnp.inf, logits)
    m1 = jnp.max(masked, axis=1, keepdims=True)
    is1 = (masked == m1).astype(jnp.float32)
    i1 = 7.0 - jnp.max(is1 * (7.0 - iota) - (1.0 - is1) * 1e9, axis=1, keepdims=True)
    oh1 = (iota == i1).astype(jnp.float32)
    w0 = 1.0 / (1.0 + jnp.exp(m1 - m0))
    w1 = 1.0 - w0
    we_ref[...] = oh0 * w0 + oh1 * w1
    p = jnp.exp(logits - m0)
    p = p / jnp.sum(p, axis=1, keepdims=True)
    avg_prob = jnp.mean(p, axis=0)
    avg_frac = jnp.mean(oh0, axis=0)
    aux = (_AUX_W * _E) * jnp.sum(avg_prob * avg_frac)
    aux_ref[...] = jnp.broadcast_to(aux, (1, 128))


def _dense_body(x_ref, w1_ref, b1_ref, w2_ref, b2_ref, we_ref, out_ref):
    e = pl.program_id(1)
    j = pl.program_id(2)
    xb = x_ref[...]
    h = jnp.dot(xb, w1_ref[0], preferred_element_type=jnp.float32) + b1_ref[0]
    h = 0.5 * h * (1.0 + jax.lax.erf(h * 0.7071067811865476))
    part = jnp.dot(h, w2_ref[0], preferred_element_type=jnp.float32)
    wcol = we_ref[0, 0]

    @pl.when(jnp.logical_and(e == 0, j == 0))
    def _():
        out_ref[...] = jnp.zeros_like(out_ref)

    @pl.when(j == 0)
    def _():
        out_ref[...] += wcol * b2_ref[0]

    out_ref[...] += wcol * part


@jax.jit
def kernel(x, Wr, W1, b1, W2, b2):
    Bz, Tz, D = x.shape
    x_flat = x.reshape(-1, D)

    wr_pad = jnp.pad(Wr, ((0, 0), (0, 128 - _E)))

    we, aux = pl.pallas_call(
        _router_body,
        out_shape=[
            jax.ShapeDtypeStruct((_N, 128), jnp.float32),
            jax.ShapeDtypeStruct((1, 128), jnp.float32),
        ],
    )(x_flat, wr_pad)

    we3 = we.T[:_E].reshape(_E, _NT, _TB, 1)

    out = pl.pallas_call(
        _dense_body,
        grid=(_NT, _E, _NF),
        in_specs=[
            pl.BlockSpec((_TB, _D), lambda t, e, j: (t, 0)),
            pl.BlockSpec((1, _D, _FC), lambda t, e, j: (e, 0, j)),
            pl.BlockSpec((1, 1, _FC), lambda t, e, j: (e, 0, j)),
            pl.BlockSpec((1, _FC, _D), lambda t, e, j: (e, j, 0)),
            pl.BlockSpec((1, 1, _D), lambda t, e, j: (e, 0, 0)),
            pl.BlockSpec((1, 1, _TB, 1), lambda t, e, j: (e, t, 0, 0)),
        ],
        out_specs=pl.BlockSpec((_TB, _D), lambda t, e, j: (t, 0)),
        out_shape=jax.ShapeDtypeStruct((_N, _D), jnp.float32),
        compiler_params=pltpu.CompilerParams(
            dimension_semantics=("parallel", "arbitrary", "arbitrary"),
        ),
    )(x_flat, W1, b1.reshape(_E, 1, _F), W2, b2.reshape(_E, 1, _D), we3)

    return out.reshape(Bz, Tz, D), aux[0, 0]
